# baseline (speedup 1.0000x reference)
import functools

import jax
import jax.numpy as jnp
from jax import lax
from jax.experimental import pallas as pl
from jax.experimental.pallas import tpu as pltpu
from jax.experimental.pallas import tpu_sc as plsc

B, N, S, D, H, C, E, DG = 4, 10000, 96, 128, 4, 32, 160000, 128
HC = H * C
NP = 10240
NB = 640
EK = 48
NCHUNK = 212
EPT = EK * NCHUNK
EP = 16 * EPT
TRASH = 10200
AW = 16


def _pre_body(xenc_ref, mask_ref, wq_ref, bq_ref, wmt_ref, bm_ref, wgt_ref,
              msrc_ref, mdst_ref, xp_ref, asrc_ref, adst_ref):
    xe = xenc_ref[0]
    q = jnp.dot(wq_ref[...], xe, preferred_element_type=jnp.float32)
    x = (q + bq_ref[...] + bm_ref[...][None, :]
         + mask_ref[0] * wmt_ref[0][None, :])
    xp = jnp.dot(x, wgt_ref[...], preferred_element_type=jnp.float32)
    xp_ref[...] = xp
    asrc_ref[...] = jnp.dot(xp, msrc_ref[...], precision=lax.Precision.HIGHEST,
                            preferred_element_type=jnp.float32)
    adst_ref[...] = jnp.dot(xp, mdst_ref[...], precision=lax.Precision.HIGHEST,
                            preferred_element_type=jnp.float32)


def _run_pre(x_enc, mask3, wq, bq2, wmt, bm, wgt, msrc, mdst):
    grid = (B, NP // NB)
    return pl.pallas_call(
        _pre_body,
        grid=grid,
        in_specs=[
            pl.BlockSpec((1, S, D), lambda b, n: (b, 0, 0)),
            pl.BlockSpec((1, NB, 1), lambda b, n: (b, n, 0)),
            pl.BlockSpec((NB, S), lambda b, n: (n, 0)),
            pl.BlockSpec((NB, 1), lambda b, n: (n, 0)),
            pl.BlockSpec((1, D), lambda b, n: (0, 0)),
            pl.BlockSpec((D,), lambda b, n: (0,)),
            pl.BlockSpec((D, HC), lambda b, n: (0, 0)),
            pl.BlockSpec((HC, AW), lambda b, n: (0, 0)),
            pl.BlockSpec((HC, AW), lambda b, n: (0, 0)),
        ],
        out_specs=[
            pl.BlockSpec((NB, HC), lambda b, n: (b * (NP // NB) + n, 0)),
            pl.BlockSpec((NB, AW), lambda b, n: (b * (NP // NB) + n, 0)),
            pl.BlockSpec((NB, AW), lambda b, n: (b * (NP // NB) + n, 0)),
        ],
        out_shape=[
            jax.ShapeDtypeStruct((B * NP, HC), jnp.float32),
            jax.ShapeDtypeStruct((B * NP, AW), jnp.float32),
            jax.ShapeDtypeStruct((B * NP, AW), jnp.float32),
        ],
    )(x_enc, mask3, wq, bq2, wmt, bm, wgt, msrc, mdst)


def _sc_body(xp_hbm, asrc_hbm, adst_hbm, idx3_hbm,
             acc_out, den_out,
             rows0, rows1, rows2, rows3, a10, a11, a12, a13,
             a20, a21, a22, a23, idx0, idx1, sidx0, sidx1,
             acc_s, den_s,
             semg0, semg1, semg2, semg3, sems0, sems1, semi0, semi1):
    c = lax.axis_index("c")
    s = lax.axis_index("s")
    zero16 = jnp.zeros((16,), jnp.float32)
    rows = (rows0, rows1, rows2, rows3)
    a1 = (a10, a11, a12, a13)
    a2 = (a20, a21, a22, a23)
    idx = (idx0, idx1)
    sidx = (sidx0, sidx1)
    semg = (semg0, semg1, semg2, semg3)
    sems = (sems0, sems1)
    semi = (semi0, semi1)
    iota = lax.iota(jnp.int32, 16)

    def fire_gather(r, q):
        pltpu.async_copy(xp_hbm.at[idx[q].at[0]], rows[r], semg[r])
        pltpu.async_copy(asrc_hbm.at[idx[q].at[0]], a1[r], semg[r])
        pltpu.async_copy(adst_hbm.at[idx[q].at[1]], a2[r], semg[r])

    def drain_gather(r, q):
        pltpu.make_async_copy(xp_hbm.at[idx[q].at[0]], rows[r],
                              semg[r]).wait()
        pltpu.make_async_copy(asrc_hbm.at[idx[q].at[0]], a1[r],
                              semg[r]).wait()
        pltpu.make_async_copy(adst_hbm.at[idx[q].at[1]], a2[r],
                              semg[r]).wait()

    def drain_scatter(r, q):
        pltpu.make_async_copy(rows[r], acc_s.at[sidx[q]], sems[q]).wait()
        pltpu.make_async_copy(a1[r], den_s.at[sidx[q]], sems[q]).wait()

    for bi in range(2):
        b = bi * 2 + c

        def zbody(k, _):
            for j in range(HC // 16):
                rows0[k, pl.ds(j * 16, 16)] = zero16
            a10[k] = zero16
            return 0
        lax.fori_loop(0, EK, zbody, 0)
        for r in range(NB // EK):
            pltpu.sync_copy(rows0, acc_s.at[pl.ds(s * NB + r * EK, EK)])
            pltpu.sync_copy(a10, den_s.at[pl.ds(s * NB + r * EK, EK)])
        rem = NB - (NB // EK) * EK
        pltpu.sync_copy(rows0.at[pl.ds(0, rem)],
                        acc_s.at[pl.ds(s * NB + NB - rem, rem)])
        pltpu.sync_copy(a10.at[pl.ds(0, rem)],
                        den_s.at[pl.ds(s * NB + NB - rem, rem)])
        plsc.subcore_barrier()

        pltpu.sync_copy(idx3_hbm.at[b, s, 0], idx0)
        pltpu.sync_copy(idx3_hbm.at[b, s, 1], idx1)
        fire_gather(0, 0)
        fire_gather(1, 1)

        def chunk4(i, _):
            for jj in range(4):
                j = 4 * i + jj
                r = jj
                q = jj % 2
                drain_gather(r, q)

                @pl.when(j >= 2)
                def _():
                    drain_scatter((jj + 2) % 4, q)

                for t in range(EK // 16):
                    sidx[q][pl.ds(t * 16, 16)] = idx[q][2, pl.ds(t * 16, 16)]

                @pl.when(j + 2 < NCHUNK)
                def _():
                    pltpu.async_copy(idx3_hbm.at[b, s, j + 2], idx[q],
                                     semi[q])

                for g in range(EK // 16):
                    rid = iota + (g * 16)
                    for h in range(H):
                        col = jnp.full((16,), h, jnp.int32)
                        e = (plsc.load_gather(a1[r], [rid, col]) +
                             plsc.load_gather(a2[r], [rid, col]))
                        e = jnp.where(e < 0.0, e * 0.2, e)
                        plsc.store_scatter(a1[r], [rid, col], jnp.exp(e))

                def ebody(k, _):
                    exv = a1[r][k]
                    for h in range(H):
                        sc = exv[h]
                        rows[r][k, pl.ds(2 * h * 16, 16)] = (
                            rows[r][k, pl.ds(2 * h * 16, 16)] * sc)
                        rows[r][k, pl.ds((2 * h + 1) * 16, 16)] = (
                            rows[r][k, pl.ds((2 * h + 1) * 16, 16)] * sc)
                    return 0
                lax.fori_loop(0, EK, ebody, 0, unroll=2)

                pltpu.async_copy(rows[r], acc_s.at[sidx[q]], sems[q],
                                 add=True)
                pltpu.async_copy(a1[r], den_s.at[sidx[q]], sems[q],
                                 add=True)

                @pl.when(j + 2 < NCHUNK)
                def _():
                    pltpu.make_async_copy(idx3_hbm.at[b, s, j + 2], idx[q],
                                          semi[q]).wait()
                    fire_gather((jj + 2) % 4, q)
            return 0
        lax.fori_loop(0, NCHUNK // 4, chunk4, 0)
        drain_scatter((NCHUNK - 2) % 4, 0)
        drain_scatter((NCHUNK - 1) % 4, 1)
        plsc.subcore_barrier()

        bo = b * NP + s * NB
        pltpu.sync_copy(acc_s.at[pl.ds(s * NB, NB)],
                        acc_out.at[pl.ds(bo, NB)])
        pltpu.sync_copy(den_s.at[pl.ds(s * NB, NB)],
                        den_out.at[pl.ds(bo, NB)])
        plsc.subcore_barrier()


def _run_sc(xp_flat, asrc_flat, adst_flat, idx3):
    mesh = plsc.VectorSubcoreMesh(core_axis_name="c", subcore_axis_name="s")
    fn = pl.kernel(
        _sc_body,
        out_type=[
            jax.ShapeDtypeStruct((B * NP, HC), jnp.float32),
            jax.ShapeDtypeStruct((B * NP, AW), jnp.float32),
        ],
        mesh=mesh,
        compiler_params=pltpu.CompilerParams(use_tc_tiling_on_sc=False,
                                             needs_layout_passes=False),
        scratch_types=(
            [pltpu.VMEM((EK, HC), jnp.float32)] * 4 +
            [pltpu.VMEM((EK, AW), jnp.float32)] * 4 +
            [pltpu.VMEM((EK, AW), jnp.float32)] * 4 +
            [pltpu.VMEM((3, EK), jnp.int32)] * 2 +
            [pltpu.VMEM((EK,), jnp.int32)] * 2 +
            [pltpu.VMEM_SHARED((NP, HC), jnp.float32),
             pltpu.VMEM_SHARED((NP, AW), jnp.float32)] +
            [pltpu.SemaphoreType.DMA] * 8
        ),
    )
    return fn(xp_flat, asrc_flat, adst_flat, idx3)


def _post_body(acc_ref, den_ref, xp_ref, asrc_ref, adst_ref, wp_ref, bp_ref,
               wtt_ref, bt_ref, bg_ref, eexp_ref, out_ref):
    ni = pl.program_id(1)
    a = asrc_ref[...] + adst_ref[...]
    aF = jnp.dot(a, eexp_ref[...], precision=lax.Precision.HIGHEST,
                 preferred_element_type=jnp.float32)
    sF = jnp.exp(jnp.where(aF < 0.0, aF * 0.2, aF))
    denF = jnp.dot(den_ref[...], eexp_ref[...],
                   precision=lax.Precision.HIGHEST,
                   preferred_element_type=jnp.float32)
    g = ((acc_ref[...] + sF * xp_ref[...]) / (denF + sF + 1e-16)
         + bg_ref[...][None, :])
    rid = ni * NB + jax.lax.broadcasted_iota(jnp.int32, (NB, 1), 0)
    g = jnp.where(rid < N, g, 0.0)
    contrib = jnp.dot(wp_ref[...], g, preferred_element_type=jnp.float32)

    @pl.when(ni == 0)
    def _():
        out_ref[0] = jnp.zeros_like(out_ref[0])
    out_ref[0] += contrib

    @pl.when(ni == NP // NB - 1)
    def _():
        o = out_ref[0] + bp_ref[...][:, None]
        out_ref[0] = jnp.dot(o, wtt_ref[...],
                             preferred_element_type=jnp.float32) + bt_ref[...][None, :]


def _run_post(acc, den, xp, asrc, adst, wp_p, bp, wtt, bt, bg, eexp):
    grid = (B, NP // NB)
    nblk = NP // NB
    return pl.pallas_call(
        _post_body,
        grid=grid,
        in_specs=[
            pl.BlockSpec((NB, HC), lambda b, n: (b * nblk + n, 0)),
            pl.BlockSpec((NB, AW), lambda b, n: (b * nblk + n, 0)),
            pl.BlockSpec((NB, HC), lambda b, n: (b * nblk + n, 0)),
            pl.BlockSpec((NB, AW), lambda b, n: (b * nblk + n, 0)),
            pl.BlockSpec((NB, AW), lambda b, n: (b * nblk + n, 0)),
            pl.BlockSpec((S, NB), lambda b, n: (0, n)),
            pl.BlockSpec((S,), lambda b, n: (0,)),
            pl.BlockSpec((HC, DG), lambda b, n: (0, 0)),
            pl.BlockSpec((DG,), lambda b, n: (0,)),
            pl.BlockSpec((HC,), lambda b, n: (0,)),
            pl.BlockSpec((AW, HC), lambda b, n: (0, 0)),
        ],
        out_specs=pl.BlockSpec((1, S, DG), lambda b, n: (b, 0, 0)),
        out_shape=jax.ShapeDtypeStruct((B, S, DG), jnp.float32),
    )(acc, den, xp, asrc, adst, wp_p, bp, wtt, bt, bg, eexp)


def kernel(x_enc, mask, edge_index, Wq, bq, Wm, bm, Wg, att_src, att_dst,
           bg, Wp, bp, Wt, bt):
    wp_p = jnp.pad(Wp, ((0, 0), (0, NP - N)))
    wmt = Wm.reshape(1, D)
    wgt = Wg.T
    wtt = Wt.T
    hsel = (jnp.arange(HC)[:, None] // C) == jnp.arange(AW)[None, :]
    msrc = jnp.where(hsel, att_src.reshape(HC, 1), 0.0)
    mdst = jnp.where(hsel, att_dst.reshape(HC, 1), 0.0)
    eexp = jnp.where(hsel.T, 1.0, 0.0)

    xp, asrc, adst = _run_pre(x_enc, mask[:, :, None], Wq, bq[:, None],
                              wmt, bm, wgt, msrc, mdst)

    src0 = jnp.concatenate(
        [edge_index[0], jnp.zeros((EP - E,), edge_index.dtype)])
    dst0 = jnp.concatenate(
        [edge_index[1], jnp.full((EP - E,), TRASH, edge_index.dtype)])
    offs = (jnp.arange(B, dtype=edge_index.dtype) * NP)[:, None]
    srcg = (src0[None, :] + offs).reshape(B, 16, NCHUNK, EK)
    dstg = (dst0[None, :] + offs).reshape(B, 16, NCHUNK, EK)
    dstl = jnp.broadcast_to(dst0.reshape(1, 16, NCHUNK, EK),
                            (B, 16, NCHUNK, EK))
    idx3 = jnp.stack([srcg, dstg, dstl], axis=3)

    acc, den = _run_sc(xp, asrc, adst, idx3)

    return _run_post(acc, den, xp, asrc, adst, wp_p, bp, wtt, bt, bg, eexp)


# speedup vs baseline: 1.0154x; 1.0154x over previous
import functools

import jax
import jax.numpy as jnp
from jax import lax
from jax.experimental import pallas as pl
from jax.experimental.pallas import tpu as pltpu
from jax.experimental.pallas import tpu_sc as plsc

B, N, S, D, H, C, E, DG = 4, 10000, 96, 128, 4, 32, 160000, 128
HC = H * C
NP = 10240
NB = 640
EK = 48
NCHUNK = 212
EPT = EK * NCHUNK
EP = 16 * EPT
TRASH = 10200
AW = 16
XW = HC + AW


def _pre_body(xenc_ref, mask_ref, wq_ref, bq_ref, wmt_ref, bm_ref, wgt_ref,
              msrc_ref, mdst_ref, xpa_ref, adst_ref):
    xe = xenc_ref[0]
    q = jnp.dot(wq_ref[...], xe, preferred_element_type=jnp.float32)
    x = (q + bq_ref[...] + bm_ref[...][None, :]
         + mask_ref[0] * wmt_ref[0][None, :])
    xp = jnp.dot(x, wgt_ref[...], preferred_element_type=jnp.float32)
    asrc = jnp.dot(xp, msrc_ref[...], precision=lax.Precision.HIGHEST,
                   preferred_element_type=jnp.float32)
    xpa_ref[...] = jnp.concatenate([xp, asrc], axis=1)
    adst_ref[...] = jnp.dot(xp, mdst_ref[...], precision=lax.Precision.HIGHEST,
                            preferred_element_type=jnp.float32)


def _run_pre(x_enc, mask3, wq, bq2, wmt, bm, wgt, msrc, mdst):
    grid = (B, NP // NB)
    return pl.pallas_call(
        _pre_body,
        grid=grid,
        in_specs=[
            pl.BlockSpec((1, S, D), lambda b, n: (b, 0, 0)),
            pl.BlockSpec((1, NB, 1), lambda b, n: (b, n, 0)),
            pl.BlockSpec((NB, S), lambda b, n: (n, 0)),
            pl.BlockSpec((NB, 1), lambda b, n: (n, 0)),
            pl.BlockSpec((1, D), lambda b, n: (0, 0)),
            pl.BlockSpec((D,), lambda b, n: (0,)),
            pl.BlockSpec((D, HC), lambda b, n: (0, 0)),
            pl.BlockSpec((HC, AW), lambda b, n: (0, 0)),
            pl.BlockSpec((HC, AW), lambda b, n: (0, 0)),
        ],
        out_specs=[
            pl.BlockSpec((NB, XW), lambda b, n: (b * (NP // NB) + n, 0)),
            pl.BlockSpec((NB, AW), lambda b, n: (b * (NP // NB) + n, 0)),
        ],
        out_shape=[
            jax.ShapeDtypeStruct((B * NP, XW), jnp.float32),
            jax.ShapeDtypeStruct((B * NP, AW), jnp.float32),
        ],
    )(x_enc, mask3, wq, bq2, wmt, bm, wgt, msrc, mdst)


def _sc_body(xpa_hbm, adst_hbm, idx3_hbm,
             acc_out,
             rows0, rows1, rows2, rows3, a20, a21, a22, a23,
             idx0, idx1, sidx0, sidx1,
             acc_s,
             semg0, semg1, semg2, semg3, sems0, sems1, semi0, semi1):
    c = lax.axis_index("c")
    s = lax.axis_index("s")
    zero16 = jnp.zeros((16,), jnp.float32)
    rows = (rows0, rows1, rows2, rows3)
    a2 = (a20, a21, a22, a23)
    idx = (idx0, idx1)
    sidx = (sidx0, sidx1)
    semg = (semg0, semg1, semg2, semg3)
    sems = (sems0, sems1)
    semi = (semi0, semi1)
    iota = lax.iota(jnp.int32, 16)

    def fire_gather(r, q):
        pltpu.async_copy(xpa_hbm.at[idx[q].at[0]], rows[r], semg[r])
        pltpu.async_copy(adst_hbm.at[idx[q].at[1]], a2[r], semg[r])

    def drain_gather(r, q):
        pltpu.make_async_copy(xpa_hbm.at[idx[q].at[0]], rows[r],
                              semg[r]).wait()
        pltpu.make_async_copy(adst_hbm.at[idx[q].at[1]], a2[r],
                              semg[r]).wait()

    def drain_scatter(r, q):
        pltpu.make_async_copy(rows[r], acc_s.at[sidx[q]], sems[q]).wait()

    for bi in range(2):
        b = bi * 2 + c

        def zbody(k, _):
            for j in range(XW // 16):
                rows0[k, pl.ds(j * 16, 16)] = zero16
            return 0
        lax.fori_loop(0, EK, zbody, 0)
        for r in range(NB // EK):
            pltpu.sync_copy(rows0, acc_s.at[pl.ds(s * NB + r * EK, EK)])
        rem = NB - (NB // EK) * EK
        pltpu.sync_copy(rows0.at[pl.ds(0, rem)],
                        acc_s.at[pl.ds(s * NB + NB - rem, rem)])
        plsc.subcore_barrier()

        pltpu.sync_copy(idx3_hbm.at[b, s, 0], idx0)
        pltpu.sync_copy(idx3_hbm.at[b, s, 1], idx1)
        fire_gather(0, 0)
        fire_gather(1, 1)

        def chunk4(i, _):
            for jj in range(4):
                j = 4 * i + jj
                r = jj
                q = jj % 2
                drain_gather(r, q)

                @pl.when(j >= 2)
                def _():
                    drain_scatter((jj + 2) % 4, q)

                for t in range(EK // 16):
                    sidx[q][pl.ds(t * 16, 16)] = idx[q][2, pl.ds(t * 16, 16)]

                @pl.when(j + 2 < NCHUNK)
                def _():
                    pltpu.async_copy(idx3_hbm.at[b, s, j + 2], idx[q],
                                     semi[q])

                for g in range(EK // 16):
                    rid = iota + (g * 16)
                    for h in range(H):
                        cola = jnp.full((16,), HC + h, jnp.int32)
                        col = jnp.full((16,), h, jnp.int32)
                        e = (plsc.load_gather(rows[r], [rid, cola]) +
                             plsc.load_gather(a2[r], [rid, col]))
                        e = jnp.where(e < 0.0, e * 0.2, e)
                        plsc.store_scatter(rows[r], [rid, cola], jnp.exp(e))

                def ebody(k, _):
                    exv = rows[r][k, pl.ds(HC, 16)]
                    for h in range(H):
                        sc = exv[h]
                        rows[r][k, pl.ds(2 * h * 16, 16)] = (
                            rows[r][k, pl.ds(2 * h * 16, 16)] * sc)
                        rows[r][k, pl.ds((2 * h + 1) * 16, 16)] = (
                            rows[r][k, pl.ds((2 * h + 1) * 16, 16)] * sc)
                    return 0
                lax.fori_loop(0, EK, ebody, 0, unroll=2)

                pltpu.async_copy(rows[r], acc_s.at[sidx[q]], sems[q],
                                 add=True)

                @pl.when(j + 2 < NCHUNK)
                def _():
                    pltpu.make_async_copy(idx3_hbm.at[b, s, j + 2], idx[q],
                                          semi[q]).wait()
                    fire_gather((jj + 2) % 4, q)
            return 0
        lax.fori_loop(0, NCHUNK // 4, chunk4, 0)
        drain_scatter((NCHUNK - 2) % 4, 0)
        drain_scatter((NCHUNK - 1) % 4, 1)
        plsc.subcore_barrier()

        bo = b * NP + s * NB
        pltpu.sync_copy(acc_s.at[pl.ds(s * NB, NB)],
                        acc_out.at[pl.ds(bo, NB)])
        plsc.subcore_barrier()


def _run_sc(xpa_flat, adst_flat, idx3):
    mesh = plsc.VectorSubcoreMesh(core_axis_name="c", subcore_axis_name="s")
    fn = pl.kernel(
        _sc_body,
        out_type=jax.ShapeDtypeStruct((B * NP, XW), jnp.float32),
        mesh=mesh,
        compiler_params=pltpu.CompilerParams(use_tc_tiling_on_sc=False,
                                             needs_layout_passes=False),
        scratch_types=(
            [pltpu.VMEM((EK, XW), jnp.float32)] * 4 +
            [pltpu.VMEM((EK, AW), jnp.float32)] * 4 +
            [pltpu.VMEM((3, EK), jnp.int32)] * 2 +
            [pltpu.VMEM((EK,), jnp.int32)] * 2 +
            [pltpu.VMEM_SHARED((NP, XW), jnp.float32)] +
            [pltpu.SemaphoreType.DMA] * 8
        ),
    )
    return fn(xpa_flat, adst_flat, idx3)


def _post_body(acc_ref, xpa_ref, adst_ref, wp_ref, bp_ref,
               wtt_ref, bt_ref, bg_ref, eexp_ref, out_ref):
    ni = pl.program_id(1)
    xpa = xpa_ref[...]
    accf = acc_ref[...]
    a = xpa[:, HC:] + adst_ref[...]
    aF = jnp.dot(a, eexp_ref[...], precision=lax.Precision.HIGHEST,
                 preferred_element_type=jnp.float32)
    sF = jnp.exp(jnp.where(aF < 0.0, aF * 0.2, aF))
    denF = jnp.dot(accf[:, HC:], eexp_ref[...],
                   precision=lax.Precision.HIGHEST,
                   preferred_element_type=jnp.float32)
    g = ((accf[:, :HC] + sF * xpa[:, :HC]) / (denF + sF + 1e-16)
         + bg_ref[...][None, :])
    rid = ni * NB + jax.lax.broadcasted_iota(jnp.int32, (NB, 1), 0)
    g = jnp.where(rid < N, g, 0.0)
    contrib = jnp.dot(wp_ref[...], g, preferred_element_type=jnp.float32)

    @pl.when(ni == 0)
    def _():
        out_ref[0] = jnp.zeros_like(out_ref[0])
    out_ref[0] += contrib

    @pl.when(ni == NP // NB - 1)
    def _():
        o = out_ref[0] + bp_ref[...][:, None]
        out_ref[0] = jnp.dot(o, wtt_ref[...],
                             preferred_element_type=jnp.float32) + bt_ref[...][None, :]


def _run_post(acc, xpa, adst, wp_p, bp, wtt, bt, bg, eexp):
    grid = (B, NP // NB)
    nblk = NP // NB
    return pl.pallas_call(
        _post_body,
        grid=grid,
        in_specs=[
            pl.BlockSpec((NB, XW), lambda b, n: (b * nblk + n, 0)),
            pl.BlockSpec((NB, XW), lambda b, n: (b * nblk + n, 0)),
            pl.BlockSpec((NB, AW), lambda b, n: (b * nblk + n, 0)),
            pl.BlockSpec((S, NB), lambda b, n: (0, n)),
            pl.BlockSpec((S,), lambda b, n: (0,)),
            pl.BlockSpec((HC, DG), lambda b, n: (0, 0)),
            pl.BlockSpec((DG,), lambda b, n: (0,)),
            pl.BlockSpec((HC,), lambda b, n: (0,)),
            pl.BlockSpec((AW, HC), lambda b, n: (0, 0)),
        ],
        out_specs=pl.BlockSpec((1, S, DG), lambda b, n: (b, 0, 0)),
        out_shape=jax.ShapeDtypeStruct((B, S, DG), jnp.float32),
    )(acc, xpa, adst, wp_p, bp, wtt, bt, bg, eexp)


def kernel(x_enc, mask, edge_index, Wq, bq, Wm, bm, Wg, att_src, att_dst,
           bg, Wp, bp, Wt, bt):
    wp_p = jnp.pad(Wp, ((0, 0), (0, NP - N)))
    wmt = Wm.reshape(1, D)
    wgt = Wg.T
    wtt = Wt.T
    hsel = (jnp.arange(HC)[:, None] // C) == jnp.arange(AW)[None, :]
    msrc = jnp.where(hsel, att_src.reshape(HC, 1), 0.0)
    mdst = jnp.where(hsel, att_dst.reshape(HC, 1), 0.0)
    eexp = jnp.where(hsel.T, 1.0, 0.0)

    xpa, adst = _run_pre(x_enc, mask[:, :, None], Wq, bq[:, None],
                         wmt, bm, wgt, msrc, mdst)

    src0 = jnp.concatenate(
        [edge_index[0], jnp.zeros((EP - E,), edge_index.dtype)])
    dst0 = jnp.concatenate(
        [edge_index[1], jnp.full((EP - E,), TRASH, edge_index.dtype)])
    offs = (jnp.arange(B, dtype=edge_index.dtype) * NP)[:, None]
    srcg = (src0[None, :] + offs).reshape(B, 16, NCHUNK, EK)
    dstg = (dst0[None, :] + offs).reshape(B, 16, NCHUNK, EK)
    dstl = jnp.broadcast_to(dst0.reshape(1, 16, NCHUNK, EK),
                            (B, 16, NCHUNK, EK))
    idx3 = jnp.stack([srcg, dstg, dstl], axis=3)

    acc = _run_sc(xpa, adst, idx3)

    return _run_post(acc, xpa, adst, wp_p, bp, wtt, bt, bg, eexp)


# speedup vs baseline: 1.0802x; 1.0639x over previous
import functools

import jax
import jax.numpy as jnp
from jax import lax
from jax.experimental import pallas as pl
from jax.experimental.pallas import tpu as pltpu
from jax.experimental.pallas import tpu_sc as plsc

B, N, S, D, H, C, E, DG = 4, 10000, 96, 128, 4, 32, 160000, 128
HC = H * C
NP = 10240
NB = 640
NBD = 2048
EK = 48
NCHUNK = 212
EPT = EK * NCHUNK
EP = 16 * EPT
TRASH = 10200
AW = 16
XW = HC + AW


def _pre_body(xenc_ref, mask_ref, wq_ref, bq_ref, wmt_ref, bm_ref, wgt_ref,
              msrc_ref, mdst_ref, xpa_ref, adst_ref):
    xe = xenc_ref[0]
    q = jnp.dot(wq_ref[...], xe, preferred_element_type=jnp.float32)
    x = (q + bq_ref[...] + bm_ref[...][None, :]
         + mask_ref[0] * wmt_ref[0][None, :])
    xp = jnp.dot(x, wgt_ref[...], preferred_element_type=jnp.float32)
    asrc = jnp.dot(xp, msrc_ref[...], precision=lax.Precision.HIGHEST,
                   preferred_element_type=jnp.float32)
    xpa_ref[...] = jnp.concatenate([xp, asrc], axis=1)
    adst_ref[...] = jnp.dot(xp, mdst_ref[...], precision=lax.Precision.HIGHEST,
                            preferred_element_type=jnp.float32)


def _run_pre(x_enc, mask3, wq, bq2, wmt, bm, wgt, msrc, mdst):
    grid = (NP // NBD, B)
    nblk = NP // NBD
    return pl.pallas_call(
        _pre_body,
        grid=grid,
        in_specs=[
            pl.BlockSpec((1, S, D), lambda n, b: (b, 0, 0)),
            pl.BlockSpec((1, NBD, 1), lambda n, b: (b, n, 0)),
            pl.BlockSpec((NBD, S), lambda n, b: (n, 0)),
            pl.BlockSpec((NBD, 1), lambda n, b: (n, 0)),
            pl.BlockSpec((1, D), lambda n, b: (0, 0)),
            pl.BlockSpec((D,), lambda n, b: (0,)),
            pl.BlockSpec((D, HC), lambda n, b: (0, 0)),
            pl.BlockSpec((HC, AW), lambda n, b: (0, 0)),
            pl.BlockSpec((HC, AW), lambda n, b: (0, 0)),
        ],
        out_specs=[
            pl.BlockSpec((NBD, XW), lambda n, b: (b * nblk + n, 0)),
            pl.BlockSpec((NBD, AW), lambda n, b: (b * nblk + n, 0)),
        ],
        out_shape=[
            jax.ShapeDtypeStruct((B * NP, XW), jnp.float32),
            jax.ShapeDtypeStruct((B * NP, AW), jnp.float32),
        ],
    )(x_enc, mask3, wq, bq2, wmt, bm, wgt, msrc, mdst)


def _sc_body(xpa_hbm, adst_hbm, idx3_hbm,
             acc_out,
             rows0, rows1, rows2, rows3, a20, a21, a22, a23,
             idx0, idx1, sidx0, sidx1,
             acc_s,
             semg0, semg1, semg2, semg3, sems0, sems1, semi0, semi1):
    c = lax.axis_index("c")
    s = lax.axis_index("s")
    zero16 = jnp.zeros((16,), jnp.float32)
    rows = (rows0, rows1, rows2, rows3)
    a2 = (a20, a21, a22, a23)
    idx = (idx0, idx1)
    sidx = (sidx0, sidx1)
    semg = (semg0, semg1, semg2, semg3)
    sems = (sems0, sems1)
    semi = (semi0, semi1)
    iota = lax.iota(jnp.int32, 16)

    def fire_gather(r, q):
        pltpu.async_copy(xpa_hbm.at[idx[q].at[0]], rows[r], semg[r])
        pltpu.async_copy(adst_hbm.at[idx[q].at[1]], a2[r], semg[r])

    def drain_gather(r, q):
        pltpu.make_async_copy(xpa_hbm.at[idx[q].at[0]], rows[r],
                              semg[r]).wait()
        pltpu.make_async_copy(adst_hbm.at[idx[q].at[1]], a2[r],
                              semg[r]).wait()

    def drain_scatter(r, q):
        pltpu.make_async_copy(rows[r], acc_s.at[sidx[q]], sems[q]).wait()

    for bi in range(2):
        b = bi * 2 + c

        def zbody(k, _):
            for j in range(XW // 16):
                rows0[k, pl.ds(j * 16, 16)] = zero16
            return 0
        lax.fori_loop(0, EK, zbody, 0)
        for r in range(NB // EK):
            pltpu.sync_copy(rows0, acc_s.at[pl.ds(s * NB + r * EK, EK)])
        rem = NB - (NB // EK) * EK
        pltpu.sync_copy(rows0.at[pl.ds(0, rem)],
                        acc_s.at[pl.ds(s * NB + NB - rem, rem)])
        plsc.subcore_barrier()

        pltpu.sync_copy(idx3_hbm.at[b, s, 0], idx0)
        pltpu.sync_copy(idx3_hbm.at[b, s, 1], idx1)
        fire_gather(0, 0)
        fire_gather(1, 1)

        def chunk4(i, _):
            for jj in range(4):
                j = 4 * i + jj
                r = jj
                q = jj % 2
                drain_gather(r, q)

                @pl.when(j >= 2)
                def _():
                    drain_scatter((jj + 2) % 4, q)

                for t in range(EK // 16):
                    sidx[q][pl.ds(t * 16, 16)] = idx[q][2, pl.ds(t * 16, 16)]

                @pl.when(j + 2 < NCHUNK)
                def _():
                    pltpu.async_copy(idx3_hbm.at[b, s, j + 2], idx[q],
                                     semi[q])

                for g in range(EK // 16):
                    rid = iota + (g * 16)
                    for h in range(H):
                        cola = jnp.full((16,), HC + h, jnp.int32)
                        col = jnp.full((16,), h, jnp.int32)
                        e = (plsc.load_gather(rows[r], [rid, cola]) +
                             plsc.load_gather(a2[r], [rid, col]))
                        e = jnp.where(e < 0.0, e * 0.2, e)
                        plsc.store_scatter(rows[r], [rid, cola], jnp.exp(e))

                def ebody(k, _):
                    exv = rows[r][k, pl.ds(HC, 16)]
                    for h in range(H):
                        sc = exv[h]
                        rows[r][k, pl.ds(2 * h * 16, 16)] = (
                            rows[r][k, pl.ds(2 * h * 16, 16)] * sc)
                        rows[r][k, pl.ds((2 * h + 1) * 16, 16)] = (
                            rows[r][k, pl.ds((2 * h + 1) * 16, 16)] * sc)
                    return 0
                lax.fori_loop(0, EK, ebody, 0, unroll=2)

                pltpu.async_copy(rows[r], acc_s.at[sidx[q]], sems[q],
                                 add=True)

                @pl.when(j + 2 < NCHUNK)
                def _():
                    pltpu.make_async_copy(idx3_hbm.at[b, s, j + 2], idx[q],
                                          semi[q]).wait()
                    fire_gather((jj + 2) % 4, q)
            return 0
        lax.fori_loop(0, NCHUNK // 4, chunk4, 0)
        drain_scatter((NCHUNK - 2) % 4, 0)
        drain_scatter((NCHUNK - 1) % 4, 1)
        plsc.subcore_barrier()

        bo = b * NP + s * NB
        pltpu.sync_copy(acc_s.at[pl.ds(s * NB, NB)],
                        acc_out.at[pl.ds(bo, NB)])
        plsc.subcore_barrier()


def _run_sc(xpa_flat, adst_flat, idx3):
    mesh = plsc.VectorSubcoreMesh(core_axis_name="c", subcore_axis_name="s")
    fn = pl.kernel(
        _sc_body,
        out_type=jax.ShapeDtypeStruct((B * NP, XW), jnp.float32),
        mesh=mesh,
        compiler_params=pltpu.CompilerParams(use_tc_tiling_on_sc=False,
                                             needs_layout_passes=False),
        scratch_types=(
            [pltpu.VMEM((EK, XW), jnp.float32)] * 4 +
            [pltpu.VMEM((EK, AW), jnp.float32)] * 4 +
            [pltpu.VMEM((3, EK), jnp.int32)] * 2 +
            [pltpu.VMEM((EK,), jnp.int32)] * 2 +
            [pltpu.VMEM_SHARED((NP, XW), jnp.float32)] +
            [pltpu.SemaphoreType.DMA] * 8
        ),
    )
    return fn(xpa_flat, adst_flat, idx3)


def _post_body(acc_ref, xpa_ref, adst_ref, wp_ref, bp_ref,
               wtt_ref, bt_ref, bg_ref, eexp_ref, out_ref):
    ni = pl.program_id(1)
    xpa = xpa_ref[...]
    accf = acc_ref[...]
    a = xpa[:, HC:] + adst_ref[...]
    aF = jnp.dot(a, eexp_ref[...], precision=lax.Precision.HIGHEST,
                 preferred_element_type=jnp.float32)
    sF = jnp.exp(jnp.where(aF < 0.0, aF * 0.2, aF))
    denF = jnp.dot(accf[:, HC:], eexp_ref[...],
                   precision=lax.Precision.HIGHEST,
                   preferred_element_type=jnp.float32)
    g = ((accf[:, :HC] + sF * xpa[:, :HC]) / (denF + sF + 1e-16)
         + bg_ref[...][None, :])
    rid = ni * NBD + jax.lax.broadcasted_iota(jnp.int32, (NBD, 1), 0)
    g = jnp.where(rid < N, g, 0.0)
    contrib = jnp.dot(wp_ref[...], g, preferred_element_type=jnp.float32)

    @pl.when(ni == 0)
    def _():
        out_ref[0] = jnp.zeros_like(out_ref[0])
    out_ref[0] += contrib

    @pl.when(ni == NP // NBD - 1)
    def _():
        o = out_ref[0] + bp_ref[...][:, None]
        out_ref[0] = jnp.dot(o, wtt_ref[...],
                             preferred_element_type=jnp.float32) + bt_ref[...][None, :]


def _run_post(acc, xpa, adst, wp_p, bp, wtt, bt, bg, eexp):
    grid = (B, NP // NBD)
    nblk = NP // NBD
    return pl.pallas_call(
        _post_body,
        grid=grid,
        in_specs=[
            pl.BlockSpec((NBD, XW), lambda b, n: (b * nblk + n, 0)),
            pl.BlockSpec((NBD, XW), lambda b, n: (b * nblk + n, 0)),
            pl.BlockSpec((NBD, AW), lambda b, n: (b * nblk + n, 0)),
            pl.BlockSpec((S, NBD), lambda b, n: (0, n)),
            pl.BlockSpec((S,), lambda b, n: (0,)),
            pl.BlockSpec((HC, DG), lambda b, n: (0, 0)),
            pl.BlockSpec((DG,), lambda b, n: (0,)),
            pl.BlockSpec((HC,), lambda b, n: (0,)),
            pl.BlockSpec((AW, HC), lambda b, n: (0, 0)),
        ],
        out_specs=pl.BlockSpec((1, S, DG), lambda b, n: (b, 0, 0)),
        out_shape=jax.ShapeDtypeStruct((B, S, DG), jnp.float32),
    )(acc, xpa, adst, wp_p, bp, wtt, bt, bg, eexp)


def kernel(x_enc, mask, edge_index, Wq, bq, Wm, bm, Wg, att_src, att_dst,
           bg, Wp, bp, Wt, bt):
    wp_p = jnp.pad(Wp, ((0, 0), (0, NP - N)))
    wmt = Wm.reshape(1, D)
    wgt = Wg.T
    wtt = Wt.T
    hsel = (jnp.arange(HC)[:, None] // C) == jnp.arange(AW)[None, :]
    msrc = jnp.where(hsel, att_src.reshape(HC, 1), 0.0)
    mdst = jnp.where(hsel, att_dst.reshape(HC, 1), 0.0)
    eexp = jnp.where(hsel.T, 1.0, 0.0)

    xpa, adst = _run_pre(x_enc, mask[:, :, None], Wq, bq[:, None],
                         wmt, bm, wgt, msrc, mdst)

    src0 = jnp.concatenate(
        [edge_index[0], jnp.zeros((EP - E,), edge_index.dtype)])
    dst0 = jnp.concatenate(
        [edge_index[1], jnp.full((EP - E,), TRASH, edge_index.dtype)])
    offs = (jnp.arange(B, dtype=edge_index.dtype) * NP)[:, None]
    srcg = (src0[None, :] + offs).reshape(B, 16, NCHUNK, EK)
    dstg = (dst0[None, :] + offs).reshape(B, 16, NCHUNK, EK)
    dstl = jnp.broadcast_to(dst0.reshape(1, 16, NCHUNK, EK),
                            (B, 16, NCHUNK, EK))
    idx3 = jnp.stack([srcg, dstg, dstl], axis=3)

    acc = _run_sc(xpa, adst, idx3)

    return _run_post(acc, xpa, adst, wp_p, bp, wtt, bt, bg, eexp)
